# Initial kernel scaffold; baseline (speedup 1.0000x reference)
import jax
import jax.numpy as jnp
from jax.experimental import pallas as pl


def kernel(feat, efeat, edge_index, W_edge, b_edge, bias):
    raise NotImplementedError("write your pallas kernel here")


# speedup vs baseline: 53.3523x; 53.3523x over previous
import functools

import jax
import jax.numpy as jnp
from jax import lax
from jax.experimental import pallas as pl
from jax.experimental.pallas import tpu as pltpu
from jax.experimental.pallas import tpu_sc as plsc

N = 10000
E = 160000
D = 16
DA = 272

NC = 2
NS = 16
NW = NC * NS
EPW = E // NW
CHUNK = 200
NCHUNK = EPW // CHUNK
NPAD = 10240
RPT = NPAD // NS

MBLK = 1000


def _matmul_body(f_ref, w_ref, g_ref):
    g_ref[...] = jnp.dot(f_ref[...], w_ref[...],
                         preferred_element_type=jnp.float32)


def _combine_body(p_ref, f_ref, b_ref, o_ref):
    o_ref[...] = p_ref[0, :N] + p_ref[1, :N] + f_ref[...] + b_ref[...]


def _sc_body(g_hbm, ef_hbm, src_hbm, dst_hbm, out_hbm,
             src_v, dst_v, ef_v, rows_v, msg_v, zero_v, acc_sh, sem):
    cid = lax.axis_index("c")
    sid = lax.axis_index("s")
    wid = cid * NS + sid

    def zero_row(r, carry):
        zero_v[r, :] = jnp.zeros((D,), jnp.float32)
        return carry

    lax.fori_loop(0, RPT, zero_row, 0)
    pltpu.sync_copy(zero_v, acc_sh.at[pl.ds(sid * RPT, RPT)])
    plsc.subcore_barrier()

    def chunk_body(i, carry):
        base = wid * EPW + i * CHUNK
        pltpu.sync_copy(src_hbm.at[pl.ds(base, CHUNK)], src_v)
        pltpu.sync_copy(dst_hbm.at[pl.ds(base, CHUNK)], dst_v)
        pltpu.sync_copy(ef_hbm.at[pl.ds(base, CHUNK), :], ef_v)
        pltpu.async_copy(g_hbm.at[src_v], rows_v, sem).wait()

        def edge_body(e, ecarry):
            ef_row = ef_v[e, :]
            acc = rows_v[e, pl.ds(256, D)]
            for k in range(D):
                acc = acc + ef_row[k] * rows_v[e, pl.ds(k * D, D)]
            msg_v[e, :] = acc
            return ecarry

        lax.fori_loop(0, CHUNK, edge_body, 0)
        pltpu.sync_copy(msg_v, acc_sh.at[dst_v], add=True)
        return carry

    lax.fori_loop(0, NCHUNK, chunk_body, 0)

    plsc.subcore_barrier()
    pltpu.sync_copy(acc_sh.at[pl.ds(sid * RPT, RPT)],
                    out_hbm.at[cid, pl.ds(sid * RPT, RPT), :])


_sc_call = functools.partial(
    pl.kernel,
    out_type=jax.ShapeDtypeStruct((NC, NPAD, D), jnp.float32),
    mesh=plsc.VectorSubcoreMesh(core_axis_name="c", subcore_axis_name="s"),
    scratch_types=[
        pltpu.VMEM((CHUNK,), jnp.int32),
        pltpu.VMEM((CHUNK,), jnp.int32),
        pltpu.VMEM((CHUNK, D), jnp.float32),
        pltpu.VMEM((CHUNK, DA), jnp.float32),
        pltpu.VMEM((CHUNK, D), jnp.float32),
        pltpu.VMEM((RPT, D), jnp.float32),
        pltpu.VMEM_SHARED((NPAD, D), jnp.float32),
        pltpu.SemaphoreType.DMA,
    ],
    compiler_params=pltpu.CompilerParams(use_tc_tiling_on_sc=False),
)(_sc_body)


@jax.jit
def kernel(feat, efeat, edge_index, W_edge, b_edge, bias):
    Wr = W_edge.reshape(D, D, D).transpose(1, 0, 2).reshape(D, D * D)
    Waug = jnp.concatenate([Wr, b_edge.reshape(D, D)], axis=1)

    g_aug = pl.pallas_call(
        _matmul_body,
        grid=(N // MBLK,),
        in_specs=[
            pl.BlockSpec((MBLK, D), lambda i: (i, 0)),
            pl.BlockSpec((D, DA), lambda i: (0, 0)),
        ],
        out_specs=pl.BlockSpec((MBLK, DA), lambda i: (i, 0)),
        out_shape=jax.ShapeDtypeStruct((N, DA), jnp.float32),
    )(feat, Waug)

    partials = _sc_call(g_aug, efeat, edge_index[0], edge_index[1])

    out = pl.pallas_call(
        _combine_body,
        out_shape=jax.ShapeDtypeStruct((N, D), jnp.float32),
    )(partials, feat, bias.reshape(1, D))
    return out


# speedup vs baseline: 73.6325x; 1.3801x over previous
import functools

import jax
import jax.numpy as jnp
from jax import lax
from jax.experimental import pallas as pl
from jax.experimental.pallas import tpu as pltpu
from jax.experimental.pallas import tpu_sc as plsc

N = 10000
E = 160000
D = 16
DA = 272

NC = 2
NS = 16
NW = NC * NS
EPW = E // NW
CHUNK = 100
NCHUNK = EPW // CHUNK
NPAD = 10240
RPT = NPAD // NS

MBLK = 1000


def _matmul_body(f_ref, w_ref, g_ref):
    g_ref[...] = jnp.dot(f_ref[...], w_ref[...],
                         preferred_element_type=jnp.float32)


def _combine_body(p_ref, f_ref, b_ref, o_ref):
    o_ref[...] = p_ref[0, :N] + p_ref[1, :N] + f_ref[...] + b_ref[...]


def _sc_body(g_hbm, ef_hbm, src_hbm, dst_hbm, out_hbm,
             srcall_v, dstall_v, ef_v, rows_v, msg_v, zero_v, acc_sh, sems):
    cid = lax.axis_index("c")
    sid = lax.axis_index("s")
    wid = cid * NS + sid
    wbase = wid * EPW

    pltpu.sync_copy(src_hbm.at[pl.ds(wid * NCHUNK, NCHUNK), :], srcall_v)
    pltpu.sync_copy(dst_hbm.at[pl.ds(wid * NCHUNK, NCHUNK), :], dstall_v)

    def zero_row(r, carry):
        zero_v[r, :] = jnp.zeros((D,), jnp.float32)
        return carry

    lax.fori_loop(0, RPT, zero_row, 0)
    pltpu.sync_copy(zero_v, acc_sh.at[pl.ds(sid * RPT, RPT)])
    plsc.subcore_barrier()

    def issue(c, slot):
        pltpu.async_copy(ef_hbm.at[pl.ds(wbase + c * CHUNK, CHUNK), :],
                         ef_v[slot], sems[slot])
        pltpu.async_copy(g_hbm.at[srcall_v.at[c]], rows_v[slot], sems[slot])

    def wait(slot):
        pltpu.make_async_copy(ef_hbm.at[pl.ds(0, CHUNK), :],
                              ef_v[slot], sems[slot]).wait()
        pltpu.make_async_copy(g_hbm.at[pl.ds(0, CHUNK), :],
                              rows_v[slot], sems[slot]).wait()

    def compute(c, slot):
        erows = rows_v[slot]
        eef = ef_v[slot]

        def edge_body(e, ecarry):
            ef_row = eef[e, :]
            acc = erows[e, pl.ds(256, D)]
            for k in range(D):
                acc = acc + ef_row[k] * erows[e, pl.ds(k * D, D)]
            msg_v[e, :] = acc
            return ecarry

        lax.fori_loop(0, CHUNK, edge_body, 0)
        pltpu.sync_copy(msg_v, acc_sh.at[dstall_v.at[c]], add=True)

    issue(0, 0)

    def pair_body(j, carry):
        c0 = 2 * j
        wait(0)
        issue(c0 + 1, 1)
        compute(c0, 0)
        wait(1)
        issue(c0 + 2, 0)
        compute(c0 + 1, 1)
        return carry

    lax.fori_loop(0, (NCHUNK - 2) // 2, pair_body, 0)
    wait(0)
    issue(NCHUNK - 1, 1)
    compute(NCHUNK - 2, 0)
    wait(1)
    compute(NCHUNK - 1, 1)

    plsc.subcore_barrier()
    pltpu.sync_copy(acc_sh.at[pl.ds(sid * RPT, RPT)],
                    out_hbm.at[cid, pl.ds(sid * RPT, RPT), :])


_sc_call = functools.partial(
    pl.kernel,
    out_type=jax.ShapeDtypeStruct((NC, NPAD, D), jnp.float32),
    mesh=plsc.VectorSubcoreMesh(core_axis_name="c", subcore_axis_name="s"),
    scratch_types=[
        pltpu.VMEM((NCHUNK, CHUNK), jnp.int32),
        pltpu.VMEM((NCHUNK, CHUNK), jnp.int32),
        [pltpu.VMEM((CHUNK, D), jnp.float32)] * 2,
        [pltpu.VMEM((CHUNK, DA), jnp.float32)] * 2,
        pltpu.VMEM((CHUNK, D), jnp.float32),
        pltpu.VMEM((RPT, D), jnp.float32),
        pltpu.VMEM_SHARED((NPAD, D), jnp.float32),
        [pltpu.SemaphoreType.DMA] * 2,
    ],
    compiler_params=pltpu.CompilerParams(use_tc_tiling_on_sc=False),
)(_sc_body)


@jax.jit
def kernel(feat, efeat, edge_index, W_edge, b_edge, bias):
    Wr = W_edge.reshape(D, D, D).transpose(1, 0, 2).reshape(D, D * D)
    Waug = jnp.concatenate([Wr, b_edge.reshape(D, D)], axis=1)

    g_aug = pl.pallas_call(
        _matmul_body,
        grid=(N // MBLK,),
        in_specs=[
            pl.BlockSpec((MBLK, D), lambda i: (i, 0)),
            pl.BlockSpec((D, DA), lambda i: (0, 0)),
        ],
        out_specs=pl.BlockSpec((MBLK, DA), lambda i: (i, 0)),
        out_shape=jax.ShapeDtypeStruct((N, DA), jnp.float32),
    )(feat, Waug)

    src2d = edge_index[0].reshape(NW * NCHUNK, CHUNK)
    dst2d = edge_index[1].reshape(NW * NCHUNK, CHUNK)
    partials = _sc_call(g_aug, efeat, src2d, dst2d)

    out = pl.pallas_call(
        _combine_body,
        out_shape=jax.ShapeDtypeStruct((N, D), jnp.float32),
    )(partials, feat, bias.reshape(1, D))
    return out
